# Initial kernel scaffold; baseline (speedup 1.0000x reference)
import jax
import jax.numpy as jnp
from jax.experimental import pallas as pl


def kernel(x, edge_index, edge_attr, edge_weight, batch, edge_index_labeled, edge_label, W_enc, b_enc, W_init, b_init, W_edge, b_edge, W1, b1, g1, be1, W2, b2, g2, be2, eps, W_lin, b_lin, Wp1, bp1, Wp2, bp2):
    raise NotImplementedError("write your pallas kernel here")


# speedup vs baseline: 1.1427x; 1.1427x over previous
import functools

import jax
import jax.numpy as jnp
from jax import lax
from jax.experimental import pallas as pl
from jax.experimental.pallas import tpu as pltpu
from jax.experimental.pallas import tpu_sc as plsc

N = 10000
E = 160000
D = 128
DE = 16
G = 256
NL = 4
LQ = 1024

NPAD = 10240
EPAD = 163840
NCORE = 2
NSUB = 16
NW = NCORE * NSUB
ROWS_PER_SUB = NPAD // NSUB
EDGES_PER_W = EPAD // NW
ECHUNK = 128
NCHUNK = EDGES_PER_W // ECHUNK
GPW = G // NW

_MESH = dict(core_axis_name="c", subcore_axis_name="s")
_SC_PARAMS = pltpu.CompilerParams(needs_layout_passes=False)


def _enc_body(x_ref, we_ref, be_ref, wi_ref, bi_ref, xf_ref, y0_ref):
    xf = jnp.maximum(
        jnp.dot(x_ref[...], we_ref[...], preferred_element_type=jnp.float32)
        + be_ref[...], 0.0)
    xf_ref[...] = xf
    y0_ref[...] = (
        jnp.dot(xf, wi_ref[...], preferred_element_type=jnp.float32)
        + bi_ref[...])


def _encoder(x, We, be, Wi, bi):
    blk = 512
    return pl.pallas_call(
        _enc_body,
        grid=(NPAD // blk,),
        in_specs=[
            pl.BlockSpec((blk, D), lambda i: (i, 0)),
            pl.BlockSpec((D, D), lambda i: (0, 0)),
            pl.BlockSpec((1, D), lambda i: (0, 0)),
            pl.BlockSpec((D, D), lambda i: (0, 0)),
            pl.BlockSpec((1, D), lambda i: (0, 0)),
        ],
        out_specs=[pl.BlockSpec((blk, D), lambda i: (i, 0)),
                   pl.BlockSpec((blk, D), lambda i: (i, 0))],
        out_shape=[jax.ShapeDtypeStruct((NPAD, D), jnp.float32),
                   jax.ShapeDtypeStruct((NPAD, D), jnp.float32)],
    )(x, We, be, Wi, bi)


def _edge_mm_body(ea_ref, w_ref, b_ref, e_ref):
    e_ref[...] = (
        jnp.dot(ea_ref[...], w_ref[...], preferred_element_type=jnp.float32)
        + b_ref[...])


def _edge_mm(ea, W, b):
    blk = 2048
    return pl.pallas_call(
        _edge_mm_body,
        grid=(EPAD // blk,),
        in_specs=[
            pl.BlockSpec((blk, DE), lambda i: (i, 0)),
            pl.BlockSpec((DE, D), lambda i: (0, 0)),
            pl.BlockSpec((1, D), lambda i: (0, 0)),
        ],
        out_specs=pl.BlockSpec((blk, D), lambda i: (i, 0)),
        out_shape=jax.ShapeDtypeStruct((EPAD, D), jnp.float32),
    )(ea, W, b)


def _node_body(s_ref, xf_ref, p0_ref, p1_ref, w1_ref, b1_ref, w2_ref, b2_ref,
               o_ref):
    z = s_ref[0] * xf_ref[...] + p0_ref[...] + p1_ref[...]
    h = jnp.maximum(
        jnp.dot(z, w1_ref[...], preferred_element_type=jnp.float32)
        + b1_ref[...], 0.0)
    o_ref[...] = jnp.maximum(
        jnp.dot(h, w2_ref[...], preferred_element_type=jnp.float32)
        + b2_ref[...], 0.0)


def _node_mlp(s, xf, parts, W1f, b1f, W2f, b2f):
    blk = 512
    nblk = NPAD // blk
    return pl.pallas_call(
        _node_body,
        grid=(nblk,),
        in_specs=[
            pl.BlockSpec(memory_space=pltpu.SMEM),
            pl.BlockSpec((blk, D), lambda i: (i, 0)),
            pl.BlockSpec((blk, D), lambda i: (i, 0)),
            pl.BlockSpec((blk, D), lambda i, nblk=nblk: (i + nblk, 0)),
            pl.BlockSpec((D, D), lambda i: (0, 0)),
            pl.BlockSpec((1, D), lambda i: (0, 0)),
            pl.BlockSpec((D, D), lambda i: (0, 0)),
            pl.BlockSpec((1, D), lambda i: (0, 0)),
        ],
        out_specs=pl.BlockSpec((blk, D), lambda i: (i, 0)),
        out_shape=jax.ShapeDtypeStruct((NPAD, D), jnp.float32),
    )(s, xf, parts, parts, W1f, b1f, W2f, b2f)


def _head_body(p0_ref, p1_ref, p2_ref, p3_ref, p4_ref, wl_ref, bsum_ref,
               wp1_ref, bp1_ref, wp2_ref, bp2_ref, eil_ref, pred_ref):
    out = p0_ref[...] + bsum_ref[...]
    for l, pref in enumerate((p1_ref, p2_ref, p3_ref, p4_ref)):
        out = out + jnp.dot(pref[...], wl_ref[l],
                            preferred_element_type=jnp.float32)
    out = jnp.maximum(out, 0.0)
    h = jnp.maximum(
        jnp.dot(out, wp1_ref[...], preferred_element_type=jnp.float32)
        + bp1_ref[...], 0.0)
    o = (jnp.dot(h, wp2_ref[...], preferred_element_type=jnp.float32)
         + bp2_ref[...])
    ids = lax.broadcasted_iota(jnp.int32, (LQ, G), 1)
    a = eil_ref[:, 0:1]
    b = eil_ref[:, 1:2]
    onehot_a = jnp.where(ids == a, 1.0, 0.0)
    onehot_b = jnp.where(ids == b, 1.0, 0.0)
    pa = jnp.dot(onehot_a, o, preferred_element_type=jnp.float32)
    pb = jnp.dot(onehot_b, o, preferred_element_type=jnp.float32)
    pred_ref[...] = pa * pb


def _head(pools, Wl, bsum, Wp1, bp1, Wp2, bp2, eilT):
    return pl.pallas_call(
        _head_body,
        out_shape=jax.ShapeDtypeStruct((LQ, 1), jnp.float32),
    )(*pools, Wl, bsum, Wp1, bp1, Wp2, bp2, eilT)


def _pool_sc(y, batch_pad, neginf):
    @functools.partial(
        pl.kernel,
        mesh=plsc.VectorSubcoreMesh(**_MESH),
        compiler_params=_SC_PARAMS,
        out_type=jax.ShapeDtypeStruct((G, D), jnp.float32),
        scratch_types=[
            pltpu.VMEM((NPAD,), jnp.int32),
            pltpu.VMEM((64, D), jnp.float32),
            pltpu.VMEM((GPW, D), jnp.float32),
        ],
    )
    def body(y_hbm, b_hbm, ninf_hbm, out_hbm, bvec, ychunk, acc):
        wid = lax.axis_index("c") * NSUB + lax.axis_index("s")
        g0 = wid * GPW
        pltpu.sync_copy(b_hbm, bvec)
        pltpu.sync_copy(ninf_hbm, acc)

        def cbody(t, carry):
            lo, hi = carry
            v = bvec[pl.ds(t * 16, 16)]
            lo = lo + jnp.sum((v < g0).astype(jnp.int32))
            hi = hi + jnp.sum((v < g0 + GPW).astype(jnp.int32))
            return (lo, hi)

        r_lo, r_hi = lax.fori_loop(0, NPAD // 16, cbody,
                                   (jnp.int32(0), jnp.int32(0)))

        lanes = [lax.iota(jnp.int32, 16) + (k * 16) for k in range(8)]

        def chunk_body(c, _):
            rbase = c * 64
            pltpu.sync_copy(y_hbm.at[pl.ds(rbase, 64)], ychunk)
            i_lo = jnp.maximum(r_lo - rbase, 0)
            i_hi = jnp.minimum(r_hi - rbase, 64)

            def row_body(i, _):
                r = rbase + i
                gv = plsc.load_gather(bvec, [jnp.full((16,), r, jnp.int32)])
                grow = gv - g0
                off = jnp.full((16,), i, jnp.int32)
                for k in range(8):
                    yv = plsc.load_gather(ychunk, [off, lanes[k]])
                    av = plsc.load_gather(acc, [grow, lanes[k]])
                    plsc.store_scatter(acc, [grow, lanes[k]],
                                       jnp.maximum(av, yv))
                return 0

            lax.fori_loop(i_lo, i_hi, row_body, 0)
            return 0

        lax.fori_loop(r_lo // 64, (r_hi + 63) // 64, chunk_body, 0)
        pltpu.sync_copy(acc, out_hbm.at[pl.ds(g0, GPW)])

    return body(y, batch_pad, neginf)


def _msg_sc(xf, e, src, dst, w, zrows):
    @functools.partial(
        pl.kernel,
        mesh=plsc.VectorSubcoreMesh(**_MESH),
        compiler_params=_SC_PARAMS,
        out_type=jax.ShapeDtypeStruct((NCORE * NPAD, D), jnp.float32),
        scratch_types=[
            pltpu.VMEM_SHARED((NPAD, D), jnp.float32),
            pltpu.VMEM((ECHUNK,), jnp.int32),
            pltpu.VMEM((1, ECHUNK), jnp.int32),
            pltpu.VMEM((ECHUNK,), jnp.float32),
            pltpu.VMEM((ECHUNK, D), jnp.float32),
            pltpu.VMEM((ECHUNK, D), jnp.float32),
            pltpu.SemaphoreType.DMA,
        ],
    )
    def body(xf_hbm, e_hbm, src_hbm, dst_hbm, w_hbm, z_hbm, out_hbm,
             acc, srcv, dstv, wv, xrows, erows, sem):
        cid = lax.axis_index("c")
        sid = lax.axis_index("s")
        wid = cid * NSUB + sid
        row0 = sid * ROWS_PER_SUB
        pltpu.sync_copy(z_hbm, acc.at[pl.ds(row0, ROWS_PER_SUB)])
        plsc.subcore_barrier()

        e0 = wid * EDGES_PER_W
        lanes = [lax.iota(jnp.int32, 16) + (k * 16) for k in range(8)]

        def chunk(t, _):
            base = e0 + t * ECHUNK
            pltpu.sync_copy(src_hbm.at[pl.ds(base, ECHUNK)], srcv)
            pltpu.sync_copy(dst_hbm.at[pl.ds(base, ECHUNK)], dstv.at[0])
            pltpu.sync_copy(w_hbm.at[pl.ds(base, ECHUNK)], wv)
            pltpu.async_copy(xf_hbm.at[srcv], xrows, sem).wait()
            pltpu.sync_copy(e_hbm.at[pl.ds(base, ECHUNK)], erows)

            def row(r, _):
                rv = jnp.full((16,), r, jnp.int32)
                wvec = plsc.load_gather(wv, [rv])
                for k in range(8):
                    xv = plsc.load_gather(xrows, [rv, lanes[k]])
                    ev = plsc.load_gather(erows, [rv, lanes[k]])
                    mv = jnp.maximum(xv + ev, 0.0) * wvec
                    plsc.store_scatter(xrows, [rv, lanes[k]], mv)
                return 0

            lax.fori_loop(0, ECHUNK, row, 0)
            pltpu.sync_copy(xrows, acc.at[dstv.at[0]], add=True)
            return 0

        lax.fori_loop(0, NCHUNK, chunk, 0)
        plsc.subcore_barrier()
        pltpu.sync_copy(acc.at[pl.ds(row0, ROWS_PER_SUB)],
                        out_hbm.at[pl.ds(cid * NPAD + row0, ROWS_PER_SUB)])

    return body(xf, e, src, dst, w, zrows)


def kernel(x, edge_index, edge_attr, edge_weight, batch, edge_index_labeled,
           edge_label, W_enc, b_enc, W_init, b_init, W_edge, b_edge, W1, b1,
           g1, be1, W2, b2, g2, be2, eps, W_lin, b_lin, Wp1, bp1, Wp2, bp2):
    f32 = jnp.float32
    xp = jnp.pad(x, ((0, NPAD - N), (0, 0)))
    batch_pad = jnp.pad(batch, (0, NPAD - N), constant_values=G)
    src = jnp.pad(edge_index[0], (0, EPAD - E))
    dst = jnp.pad(edge_index[1], (0, EPAD - E))
    eap = jnp.pad(edge_attr, ((0, EPAD - E), (0, 0)))
    wp = jnp.pad(edge_weight, (0, EPAD - E))
    neginf = jnp.full((GPW, D), -3.0e38, f32)
    zrows = jnp.zeros((ROWS_PER_SUB, D), f32)

    xf, y0 = _encoder(xp, W_enc, b_enc.reshape(1, D), W_init,
                      b_init.reshape(1, D))
    pools = [_pool_sc(y0, batch_pad, neginf)]
    for l in range(NL):
        el = _edge_mm(eap, W_edge[l], b_edge[l].reshape(1, D))
        parts = _msg_sc(xf, el, src, dst, wp, zrows)
        W1f = W1[l] * g1[l][None, :]
        b1f = (b1[l] * g1[l] + be1[l]).reshape(1, D)
        W2f = W2[l] * g2[l][None, :]
        b2f = (b2[l] * g2[l] + be2[l]).reshape(1, D)
        s = (1.0 + eps[l]).reshape(1)
        xf = _node_mlp(s, xf, parts, W1f, b1f, W2f, b2f)
        pools.append(_pool_sc(xf, batch_pad, neginf))

    bsum = jnp.sum(b_lin, axis=0).reshape(1, D)
    eilT = edge_index_labeled.T
    pred = _head(pools, W_lin, bsum, Wp1, bp1.reshape(1, D), Wp2,
                 bp2.reshape(1, 1), eilT)
    return pred.reshape(LQ), edge_label


# speedup vs baseline: 2.0822x; 1.8221x over previous
import functools

import jax
import jax.numpy as jnp
from jax import lax
from jax.experimental import pallas as pl
from jax.experimental.pallas import tpu as pltpu
from jax.experimental.pallas import tpu_sc as plsc

N = 10000
E = 160000
D = 128
DE = 16
G = 256
NL = 4
LQ = 1024

NPAD = 10240
EPAD = 163840
NCORE = 2
NSUB = 16
NW = NCORE * NSUB
ROWS_PER_SUB = NPAD // NSUB
EDGES_PER_W = EPAD // NW
ECHUNK = 64
NCHUNK = EDGES_PER_W // ECHUNK
GPW = G // NW

_MESH = dict(core_axis_name="c", subcore_axis_name="s")
_SC_PARAMS = pltpu.CompilerParams(needs_layout_passes=False)


def _enc_body(x_ref, we_ref, be_ref, wi_ref, bi_ref, xf_ref, y0_ref):
    xf = jnp.maximum(
        jnp.dot(x_ref[...], we_ref[...], preferred_element_type=jnp.float32)
        + be_ref[...], 0.0)
    xf_ref[...] = xf
    y0_ref[...] = (
        jnp.dot(xf, wi_ref[...], preferred_element_type=jnp.float32)
        + bi_ref[...])


def _encoder(x, We, be, Wi, bi):
    blk = 512
    return pl.pallas_call(
        _enc_body,
        grid=(NPAD // blk,),
        in_specs=[
            pl.BlockSpec((blk, D), lambda i: (i, 0)),
            pl.BlockSpec((D, D), lambda i: (0, 0)),
            pl.BlockSpec((1, D), lambda i: (0, 0)),
            pl.BlockSpec((D, D), lambda i: (0, 0)),
            pl.BlockSpec((1, D), lambda i: (0, 0)),
        ],
        out_specs=[pl.BlockSpec((blk, D), lambda i: (i, 0)),
                   pl.BlockSpec((blk, D), lambda i: (i, 0))],
        out_shape=[jax.ShapeDtypeStruct((NPAD, D), jnp.float32),
                   jax.ShapeDtypeStruct((NPAD, D), jnp.float32)],
    )(x, We, be, Wi, bi)


def _edge_mm_body(ea_ref, w_ref, b_ref, e_ref):
    e_ref[...] = (
        jnp.dot(ea_ref[...], w_ref[...], preferred_element_type=jnp.float32)
        + b_ref[...])


def _edge_mm(ea, W, b):
    blk = 2048
    return pl.pallas_call(
        _edge_mm_body,
        grid=(EPAD // blk,),
        in_specs=[
            pl.BlockSpec((blk, DE), lambda i: (i, 0)),
            pl.BlockSpec((DE, D), lambda i: (0, 0)),
            pl.BlockSpec((1, D), lambda i: (0, 0)),
        ],
        out_specs=pl.BlockSpec((blk, D), lambda i: (i, 0)),
        out_shape=jax.ShapeDtypeStruct((EPAD, D), jnp.float32),
    )(ea, W, b)


def _node_body(s_ref, xf_ref, p0_ref, p1_ref, w1_ref, b1_ref, w2_ref, b2_ref,
               o_ref):
    z = s_ref[0] * xf_ref[...] + p0_ref[...] + p1_ref[...]
    h = jnp.maximum(
        jnp.dot(z, w1_ref[...], preferred_element_type=jnp.float32)
        + b1_ref[...], 0.0)
    o_ref[...] = jnp.maximum(
        jnp.dot(h, w2_ref[...], preferred_element_type=jnp.float32)
        + b2_ref[...], 0.0)


def _node_mlp(s, xf, parts, W1f, b1f, W2f, b2f):
    blk = 512
    nblk = NPAD // blk
    return pl.pallas_call(
        _node_body,
        grid=(nblk,),
        in_specs=[
            pl.BlockSpec(memory_space=pltpu.SMEM),
            pl.BlockSpec((blk, D), lambda i: (i, 0)),
            pl.BlockSpec((blk, D), lambda i: (i, 0)),
            pl.BlockSpec((blk, D), lambda i, nblk=nblk: (i + nblk, 0)),
            pl.BlockSpec((D, D), lambda i: (0, 0)),
            pl.BlockSpec((1, D), lambda i: (0, 0)),
            pl.BlockSpec((D, D), lambda i: (0, 0)),
            pl.BlockSpec((1, D), lambda i: (0, 0)),
        ],
        out_specs=pl.BlockSpec((blk, D), lambda i: (i, 0)),
        out_shape=jax.ShapeDtypeStruct((NPAD, D), jnp.float32),
    )(s, xf, parts, parts, W1f, b1f, W2f, b2f)


def _head_body(p0_ref, p1_ref, p2_ref, p3_ref, p4_ref, wl_ref, bsum_ref,
               wp1_ref, bp1_ref, wp2_ref, bp2_ref, eil_ref, pred_ref):
    out = p0_ref[...] + bsum_ref[...]
    for l, pref in enumerate((p1_ref, p2_ref, p3_ref, p4_ref)):
        out = out + jnp.dot(pref[...], wl_ref[l],
                            preferred_element_type=jnp.float32)
    out = jnp.maximum(out, 0.0)
    h = jnp.maximum(
        jnp.dot(out, wp1_ref[...], preferred_element_type=jnp.float32)
        + bp1_ref[...], 0.0)
    o = (jnp.dot(h, wp2_ref[...], preferred_element_type=jnp.float32)
         + bp2_ref[...])
    ids = lax.broadcasted_iota(jnp.int32, (LQ, G), 1)
    a = eil_ref[:, 0:1]
    b = eil_ref[:, 1:2]
    onehot_a = jnp.where(ids == a, 1.0, 0.0)
    onehot_b = jnp.where(ids == b, 1.0, 0.0)
    pa = jnp.dot(onehot_a, o, preferred_element_type=jnp.float32)
    pb = jnp.dot(onehot_b, o, preferred_element_type=jnp.float32)
    pred_ref[...] = pa * pb


def _head(pools, Wl, bsum, Wp1, bp1, Wp2, bp2, eilT):
    return pl.pallas_call(
        _head_body,
        out_shape=jax.ShapeDtypeStruct((LQ, 1), jnp.float32),
    )(*pools, Wl, bsum, Wp1, bp1, Wp2, bp2, eilT)


def _pool_sc(y, batch_pad, neginf):
    @functools.partial(
        pl.kernel,
        mesh=plsc.VectorSubcoreMesh(**_MESH),
        compiler_params=_SC_PARAMS,
        out_type=jax.ShapeDtypeStruct((G, D), jnp.float32),
        scratch_types=[
            pltpu.VMEM((NPAD,), jnp.int32),
            pltpu.VMEM((64, D), jnp.float32),
            pltpu.VMEM((GPW, D), jnp.float32),
        ],
    )
    def body(y_hbm, b_hbm, ninf_hbm, out_hbm, bvec, ychunk, acc):
        wid = lax.axis_index("c") * NSUB + lax.axis_index("s")
        g0 = wid * GPW
        pltpu.sync_copy(b_hbm, bvec)
        pltpu.sync_copy(ninf_hbm, acc)

        def cbody(t, carry):
            lo, hi = carry
            v = bvec[pl.ds(t * 16, 16)]
            lo = lo + jnp.sum((v < g0).astype(jnp.int32))
            hi = hi + jnp.sum((v < g0 + GPW).astype(jnp.int32))
            return (lo, hi)

        r_lo, r_hi = lax.fori_loop(0, NPAD // 16, cbody,
                                   (jnp.int32(0), jnp.int32(0)))

        lanes = [lax.iota(jnp.int32, 16) + (k * 16) for k in range(8)]

        def chunk_body(c, _):
            rbase = c * 64
            pltpu.sync_copy(y_hbm.at[pl.ds(rbase, 64)], ychunk)
            i_lo = jnp.maximum(r_lo - rbase, 0)
            i_hi = jnp.minimum(r_hi - rbase, 64)

            def row_body(i, _):
                r = rbase + i
                gv = plsc.load_gather(bvec, [jnp.full((16,), r, jnp.int32)])
                grow = gv - g0
                off = jnp.full((16,), i, jnp.int32)
                for k in range(8):
                    yv = plsc.load_gather(ychunk, [off, lanes[k]])
                    av = plsc.load_gather(acc, [grow, lanes[k]])
                    plsc.store_scatter(acc, [grow, lanes[k]],
                                       jnp.maximum(av, yv))
                return 0

            lax.fori_loop(i_lo, i_hi, row_body, 0)
            return 0

        lax.fori_loop(r_lo // 64, (r_hi + 63) // 64, chunk_body, 0)
        pltpu.sync_copy(acc, out_hbm.at[pl.ds(g0, GPW)])

    return body(y, batch_pad, neginf)


def _msg_sc(xf, e, idx3, zrows):
    @functools.partial(
        pl.kernel,
        mesh=plsc.VectorSubcoreMesh(**_MESH),
        compiler_params=_SC_PARAMS,
        out_type=jax.ShapeDtypeStruct((NCORE * NPAD, D), jnp.float32),
        scratch_types=[
            pltpu.VMEM_SHARED((NPAD, D), jnp.float32),
            pltpu.VMEM((3, ECHUNK), jnp.int32),
            pltpu.VMEM((3, ECHUNK), jnp.int32),
            pltpu.VMEM((3, ECHUNK), jnp.int32),
            pltpu.VMEM((3, ECHUNK), jnp.int32),
            pltpu.VMEM((ECHUNK, D), jnp.float32),
            pltpu.VMEM((ECHUNK, D), jnp.float32),
            pltpu.VMEM((ECHUNK, D), jnp.float32),
            pltpu.VMEM((ECHUNK, D), jnp.float32),
            pltpu.SemaphoreType.DMA,
            pltpu.SemaphoreType.DMA,
            pltpu.SemaphoreType.DMA,
            pltpu.SemaphoreType.DMA,
            pltpu.SemaphoreType.DMA,
            pltpu.SemaphoreType.DMA,
            pltpu.SemaphoreType.DMA,
            pltpu.SemaphoreType.DMA,
            pltpu.SemaphoreType.DMA,
            pltpu.SemaphoreType.DMA,
        ],
    )
    def body(xf_hbm, e_hbm, idx3_hbm, z_hbm, out_hbm,
             acc, ib0, ib1, ib2, ib3, xr0, xr1, er0, er1,
             gi0, gi1, gi2, gi3, gx0, gx1, ge0, ge1, ss0, ss1):
        cid = lax.axis_index("c")
        sid = lax.axis_index("s")
        wid = cid * NSUB + sid
        row0 = sid * ROWS_PER_SUB
        pltpu.sync_copy(z_hbm, acc.at[pl.ds(row0, ROWS_PER_SUB)])
        plsc.subcore_barrier()

        i0 = wid * NCHUNK
        e0 = wid * EDGES_PER_W
        ib = (ib0, ib1, ib2, ib3)
        gi = (gi0, gi1, gi2, gi3)
        xr = (xr0, xr1)
        er = (er0, er1)
        gx = (gx0, gx1)
        ge = (ge0, ge1)
        ss = (ss0, ss1)

        def start_idx(t, j):
            pltpu.async_copy(idx3_hbm.at[i0 + t], ib[j], gi[j])

        def wait_idx(t, j):
            pltpu.make_async_copy(idx3_hbm.at[i0 + t], ib[j], gi[j]).wait()

        def start_loads(t, b, j):
            pltpu.async_copy(xf_hbm.at[ib[j].at[0]], xr[b], gx[b])
            pltpu.async_copy(e_hbm.at[pl.ds(e0 + t * ECHUNK, ECHUNK)],
                             er[b], ge[b])

        def wait_loads(t, b, j):
            pltpu.make_async_copy(xf_hbm.at[ib[j].at[0]], xr[b], gx[b]).wait()
            pltpu.make_async_copy(e_hbm.at[pl.ds(e0 + t * ECHUNK, ECHUNK)],
                                  er[b], ge[b]).wait()

        def compute(b, j):
            xrb, erb, ibj = xr[b], er[b], ib[j]

            def row(r, _):
                wbits = plsc.load_gather(
                    ibj, [jnp.full((16,), 2, jnp.int32),
                          jnp.full((16,), r, jnp.int32)])
                wvec = plsc.bitcast(wbits, jnp.float32)
                for k in range(8):
                    sl = pl.ds(k * 16, 16)
                    xrb[r, sl] = jnp.maximum(xrb[r, sl] + erb[r, sl],
                                             0.0) * wvec
                return 0

            lax.fori_loop(0, ECHUNK, row, 0)

        def start_scatter(b, j):
            pltpu.async_copy(xr[b], acc.at[ib[j].at[1]], ss[b], add=True)

        def wait_scatter(b, j):
            pltpu.make_async_copy(xr[b], acc.at[ib[j].at[1]], ss[b]).wait()

        pltpu.async_copy(idx3_hbm.at[i0], ib0, gi0).wait()
        start_idx(1, 1)
        start_idx(2, 2)
        start_loads(0, 0, 0)

        NQ = NCHUNK // 4

        def quad(s, _):
            for q in range(4):
                t = 4 * s + q
                b = q & 1
                nb = (q + 1) & 1
                jq = q
                jn = (q + 1) & 3
                last = NQ - 1
                if q == 3:
                    pl.when(s < last)(lambda: wait_idx(t + 1, jn))
                else:
                    wait_idx(t + 1, jn)
                if q == 0:
                    pl.when(s > 0)(lambda: wait_scatter(nb, 3))
                else:
                    wait_scatter(nb, q - 1)
                if q == 3:
                    pl.when(s < last)(lambda: start_loads(t + 1, nb, jn))
                else:
                    start_loads(t + 1, nb, jn)
                if q == 0:
                    start_idx(t + 3, 3)
                else:
                    pl.when(s < last)(lambda: start_idx(t + 3, (q + 3) & 3))
                wait_loads(t, b, jq)
                compute(b, jq)
                start_scatter(b, jq)
            return 0

        lax.fori_loop(0, NQ, quad, 0)
        wait_scatter(1, 3)
        plsc.subcore_barrier()
        pltpu.sync_copy(acc.at[pl.ds(row0, ROWS_PER_SUB)],
                        out_hbm.at[pl.ds(cid * NPAD + row0, ROWS_PER_SUB)])

    return body(xf, e, idx3, zrows)


def kernel(x, edge_index, edge_attr, edge_weight, batch, edge_index_labeled,
           edge_label, W_enc, b_enc, W_init, b_init, W_edge, b_edge, W1, b1,
           g1, be1, W2, b2, g2, be2, eps, W_lin, b_lin, Wp1, bp1, Wp2, bp2):
    f32 = jnp.float32
    xp = jnp.pad(x, ((0, NPAD - N), (0, 0)))
    batch_pad = jnp.pad(batch, (0, NPAD - N), constant_values=G)
    src2 = jnp.pad(edge_index[0], (0, EPAD - E)).reshape(EPAD // ECHUNK,
                                                         ECHUNK)
    dst2 = jnp.pad(edge_index[1], (0, EPAD - E)).reshape(EPAD // ECHUNK,
                                                         ECHUNK)
    wbits = lax.bitcast_convert_type(
        jnp.pad(edge_weight, (0, EPAD - E)),
        jnp.int32).reshape(EPAD // ECHUNK, ECHUNK)
    idx3 = jnp.stack([src2, dst2, wbits], axis=1)
    eap = jnp.pad(edge_attr, ((0, EPAD - E), (0, 0)))
    neginf = jnp.full((GPW, D), -3.0e38, f32)
    zrows = jnp.zeros((ROWS_PER_SUB, D), f32)

    xf, y0 = _encoder(xp, W_enc, b_enc.reshape(1, D), W_init,
                      b_init.reshape(1, D))
    pools = [_pool_sc(y0, batch_pad, neginf)]
    for l in range(NL):
        el = _edge_mm(eap, W_edge[l], b_edge[l].reshape(1, D))
        parts = _msg_sc(xf, el, idx3, zrows)
        W1f = W1[l] * g1[l][None, :]
        b1f = (b1[l] * g1[l] + be1[l]).reshape(1, D)
        W2f = W2[l] * g2[l][None, :]
        b2f = (b2[l] * g2[l] + be2[l]).reshape(1, D)
        s = (1.0 + eps[l]).reshape(1)
        xf = _node_mlp(s, xf, parts, W1f, b1f, W2f, b2f)
        pools.append(_pool_sc(xf, batch_pad, neginf))

    bsum = jnp.sum(b_lin, axis=0).reshape(1, D)
    eilT = edge_index_labeled.T
    pred = _head(pools, W_lin, bsum, Wp1, bp1.reshape(1, D), Wp2,
                 bp2.reshape(1, 1), eilT)
    return pred.reshape(LQ), edge_label
